# baseline (speedup 1.0000x reference)
import functools

import jax
import jax.numpy as jnp
from jax import lax
from jax.experimental import pallas as pl
from jax.experimental.pallas import tpu as pltpu
from jax.experimental.pallas import tpu_sc as plsc

N = 10000
D = 128
E = 320000

NW = 32
EPB = 128
CPT = 80
C_SLOW = 56
C_FAST = 104
NCH = 16 * (C_SLOW + C_FAST)
EPAD = NCH * EPB
NROWS = 10112
RPT = NROWS // 16
NDEG = 10240
DPT = NDEG // 16

_mesh = plsc.VectorSubcoreMesh(core_axis_name="c", subcore_axis_name="s")


@functools.partial(
    pl.kernel,
    mesh=_mesh,
    out_type=jax.ShapeDtypeStruct((2 * NDEG,), jnp.float32),
    scratch_types=[
        pltpu.VMEM((NCH // NW, EPB), jnp.int32),
        pltpu.VMEM((EPB,), jnp.float32),
        pltpu.VMEM((DPT,), jnp.float32),
        pltpu.VMEM_SHARED((NDEG,), jnp.float32),
    ],
)
def _sc_deg(dst_hbm, out_hbm, idx_v, ones_v, zb_v, deg_sh):
    c = lax.axis_index("c")
    s = lax.axis_index("s")
    wid = c * 16 + s

    for i in range(EPB // 16):
        ones_v[pl.ds(i * 16, 16)] = jnp.ones((16,), jnp.float32)
    for i in range(DPT // 16):
        zb_v[pl.ds(i * 16, 16)] = jnp.zeros((16,), jnp.float32)
    pltpu.sync_copy(zb_v, deg_sh.at[pl.ds(s * DPT, DPT)])
    plsc.subcore_barrier()
    pltpu.sync_copy(dst_hbm.at[pl.ds(wid * CPT, CPT)], idx_v)

    def body(j, carry):
        pltpu.sync_copy(ones_v, deg_sh.at[idx_v.at[j]], add=True)
        return carry

    lax.fori_loop(0, CPT, body, 0)
    plsc.subcore_barrier()
    pltpu.sync_copy(deg_sh.at[pl.ds(s * DPT, DPT)],
                    out_hbm.at[pl.ds(c * NDEG + s * DPT, DPT)])


@functools.partial(
    pl.kernel,
    mesh=_mesh,
    out_type=jax.ShapeDtypeStruct((2, NROWS, D), jnp.float32),
    scratch_types=[
        pltpu.VMEM((C_FAST, EPB), jnp.int32),
        pltpu.VMEM((C_FAST, EPB), jnp.int32),
        pltpu.VMEM((EPB, D), jnp.float32),
        pltpu.VMEM_SHARED((NROWS, D), jnp.float32),
        pltpu.SemaphoreType.DMA,
    ],
)
def _sc_spmm(g_hbm, src_hbm, dst_hbm, out_hbm, si_v, di_v, rows_v,
             acc_sh, sem):
    c = lax.axis_index("c")
    s = lax.axis_index("s")
    wid = c * 16 + s

    def zrow(j, carry):
        for i in range(D // 16):
            rows_v[j, pl.ds(i * 16, 16)] = jnp.zeros((16,), jnp.float32)
        return carry

    lax.fori_loop(0, EPB, zrow, 0)
    for b in range(4):
        pltpu.sync_copy(rows_v, acc_sh.at[pl.ds(s * RPT + b * EPB, EPB)])
    pltpu.sync_copy(rows_v.at[pl.ds(0, RPT - 4 * EPB)],
                    acc_sh.at[pl.ds(s * RPT + 4 * EPB, RPT - 4 * EPB)])
    base = (1 - c) * s * C_SLOW + c * (16 * C_SLOW + s * C_FAST)
    nch = jnp.where(c == 0, C_SLOW, C_FAST)
    pltpu.sync_copy(src_hbm.at[pl.ds(base, C_FAST)], si_v)
    pltpu.sync_copy(dst_hbm.at[pl.ds(base, C_FAST)], di_v)
    plsc.subcore_barrier()

    def body(j, carry):
        pltpu.async_copy(g_hbm.at[di_v.at[j]], rows_v, sem).wait()
        pltpu.sync_copy(rows_v, acc_sh.at[si_v.at[j]], add=True)
        return carry

    lax.fori_loop(0, nch, body, 0)
    plsc.subcore_barrier()
    pltpu.sync_copy(acc_sh.at[pl.ds(s * RPT, RPT)],
                    out_hbm.at[c, pl.ds(s * RPT, RPT)])


_RB = 2528


def _tc_g_body(h_ref, d0_ref, d1_ref, g_ref, dis_ref):
    dis = lax.rsqrt(d0_ref[...] + d1_ref[...] + 1.0)
    g_ref[...] = h_ref[...] * dis
    dis_ref[...] = dis


def _tc_g(h_pad, d0, d1):
    grid = (NROWS // _RB,)
    return pl.pallas_call(
        _tc_g_body,
        grid=grid,
        in_specs=[
            pl.BlockSpec((_RB, D), lambda i: (i, 0)),
            pl.BlockSpec((_RB, 1), lambda i: (i, 0)),
            pl.BlockSpec((_RB, 1), lambda i: (i, 0)),
        ],
        out_specs=[
            pl.BlockSpec((_RB, D), lambda i: (i, 0)),
            pl.BlockSpec((_RB, 1), lambda i: (i, 0)),
        ],
        out_shape=[
            jax.ShapeDtypeStruct((NROWS, D), jnp.float32),
            jax.ShapeDtypeStruct((NROWS, 1), jnp.float32),
        ],
    )(h_pad, d0, d1)


def _tc_final_body(h_ref, g_ref, acc_ref, dis_ref, c1_ref, c2_ref, o_ref):
    accsum = acc_ref[0] + acc_ref[1]
    a = dis_ref[...] * (accsum + g_ref[...])
    o_ref[...] = h_ref[...] * c1_ref[...] + a * c2_ref[...]


def _tc_final(h_pad, g_pad, acc, dis_col, c1, c2):
    grid = (NROWS // _RB,)
    return pl.pallas_call(
        _tc_final_body,
        grid=grid,
        in_specs=[
            pl.BlockSpec((_RB, D), lambda i: (i, 0)),
            pl.BlockSpec((_RB, D), lambda i: (i, 0)),
            pl.BlockSpec((2, _RB, D), lambda i: (0, i, 0)),
            pl.BlockSpec((_RB, 1), lambda i: (i, 0)),
            pl.BlockSpec((1, D), lambda i: (0, 0)),
            pl.BlockSpec((1, D), lambda i: (0, 0)),
        ],
        out_specs=pl.BlockSpec((_RB, D), lambda i: (i, 0)),
        out_shape=jax.ShapeDtypeStruct((NROWS, D), jnp.float32),
    )(h_pad, g_pad, acc, dis_col, c1, c2)


def kernel(h, edge_index, k_cross_channel, K_channel_wise):
    src = edge_index[0].astype(jnp.int32)
    dst = edge_index[1].astype(jnp.int32)
    pad = jnp.full((EPAD - E,), N, jnp.int32)
    srcp = jnp.concatenate([src, pad]).reshape(NCH, EPB)
    dstp = jnp.concatenate([dst, pad]).reshape(NCH, EPB)
    h_pad = jnp.pad(h, ((0, NROWS - N), (0, 0)))

    deg_flat = _sc_deg(dstp)
    d0 = deg_flat[:NROWS, None]
    d1 = deg_flat[NDEG:NDEG + NROWS, None]
    g_pad, dis_col = _tc_g(h_pad, d0, d1)
    acc = _sc_spmm(g_pad, srcp, dstp)

    k = k_cross_channel[0]
    c1 = (2.0 - k - K_channel_wise) * 0.5
    c2 = (k + K_channel_wise) * 0.5
    out_pad = _tc_final(h_pad, g_pad, acc, dis_col, c1, c2)
    return out_pad[:N]


# speedup vs baseline: 2.7010x; 2.7010x over previous
import functools

import jax
import jax.numpy as jnp
from jax import lax
from jax.experimental import pallas as pl
from jax.experimental.pallas import tpu as pltpu
from jax.experimental.pallas import tpu_sc as plsc

N = 10000
D = 128
E = 320000

NW = 32
EPB = 128
CPT = 80
NCH = NW * CPT
EPAD = NCH * EPB
NROWS = 10112
RPT = NROWS // 16
NDEG = 10240
DPT = NDEG // 16

_mesh = plsc.VectorSubcoreMesh(core_axis_name="c", subcore_axis_name="s")


@functools.partial(
    pl.kernel,
    mesh=_mesh,
    out_type=jax.ShapeDtypeStruct((2 * NDEG,), jnp.float32),
    scratch_types=[
        pltpu.VMEM((NCH // NW, EPB), jnp.int32),
        pltpu.VMEM((EPB,), jnp.float32),
        pltpu.VMEM((DPT,), jnp.float32),
        pltpu.VMEM_SHARED((NDEG,), jnp.float32),
    ],
)
def _sc_deg(dst_hbm, out_hbm, idx_v, ones_v, zb_v, deg_sh):
    c = lax.axis_index("c")
    s = lax.axis_index("s")
    wid = c * 16 + s

    for i in range(EPB // 16):
        ones_v[pl.ds(i * 16, 16)] = jnp.ones((16,), jnp.float32)
    for i in range(DPT // 16):
        zb_v[pl.ds(i * 16, 16)] = jnp.zeros((16,), jnp.float32)
    pltpu.sync_copy(zb_v, deg_sh.at[pl.ds(s * DPT, DPT)])
    plsc.subcore_barrier()
    pltpu.sync_copy(dst_hbm.at[pl.ds(wid * CPT, CPT)], idx_v)

    def body(j, carry):
        pltpu.sync_copy(ones_v, deg_sh.at[idx_v.at[j]], add=True)
        return carry

    lax.fori_loop(0, CPT, body, 0)
    plsc.subcore_barrier()
    pltpu.sync_copy(deg_sh.at[pl.ds(s * DPT, DPT)],
                    out_hbm.at[pl.ds(c * NDEG + s * DPT, DPT)])


@functools.partial(
    pl.kernel,
    mesh=_mesh,
    out_type=jax.ShapeDtypeStruct((2, NROWS, D), jnp.float32),
    scratch_types=[
        pltpu.VMEM((CPT, EPB), jnp.int32),
        pltpu.VMEM((CPT, EPB), jnp.int32),
        pltpu.VMEM((EPB, D), jnp.float32),
        pltpu.VMEM_SHARED((NROWS, D), jnp.float32),
        pltpu.SemaphoreType.DMA,
    ],
)
def _sc_spmm(g_hbm, src_hbm, dst_hbm, out_hbm, si_v, di_v, rows_v,
             acc_sh, sem):
    c = lax.axis_index("c")
    s = lax.axis_index("s")
    wid = c * 16 + s

    def zrow(j, carry):
        for i in range(D // 16):
            rows_v[j, pl.ds(i * 16, 16)] = jnp.zeros((16,), jnp.float32)
        return carry

    lax.fori_loop(0, EPB, zrow, 0)
    for b in range(4):
        pltpu.sync_copy(rows_v, acc_sh.at[pl.ds(s * RPT + b * EPB, EPB)])
    pltpu.sync_copy(rows_v.at[pl.ds(0, RPT - 4 * EPB)],
                    acc_sh.at[pl.ds(s * RPT + 4 * EPB, RPT - 4 * EPB)])
    pltpu.sync_copy(src_hbm.at[pl.ds(wid * CPT, CPT)], si_v)
    pltpu.sync_copy(dst_hbm.at[pl.ds(wid * CPT, CPT)], di_v)
    plsc.subcore_barrier()

    def body(j, carry):
        pltpu.async_copy(g_hbm.at[di_v.at[j]], rows_v, sem).wait()
        pltpu.sync_copy(rows_v, acc_sh.at[si_v.at[j]], add=True)
        return carry

    lax.fori_loop(0, CPT, body, 0)
    plsc.subcore_barrier()
    pltpu.sync_copy(acc_sh.at[pl.ds(s * RPT, RPT)],
                    out_hbm.at[c, pl.ds(s * RPT, RPT)])


_RB = 2528


def _tc_g_body(h_ref, d0_ref, d1_ref, g_ref, dis_ref):
    dis = lax.rsqrt(d0_ref[...] + d1_ref[...] + 1.0)
    g_ref[...] = h_ref[...] * dis
    dis_ref[...] = dis


def _tc_g(h_pad, d0, d1):
    grid = (NROWS // _RB,)
    return pl.pallas_call(
        _tc_g_body,
        grid=grid,
        in_specs=[
            pl.BlockSpec((_RB, D), lambda i: (i, 0)),
            pl.BlockSpec((_RB, 1), lambda i: (i, 0)),
            pl.BlockSpec((_RB, 1), lambda i: (i, 0)),
        ],
        out_specs=[
            pl.BlockSpec((_RB, D), lambda i: (i, 0)),
            pl.BlockSpec((_RB, 1), lambda i: (i, 0)),
        ],
        out_shape=[
            jax.ShapeDtypeStruct((NROWS, D), jnp.float32),
            jax.ShapeDtypeStruct((NROWS, 1), jnp.float32),
        ],
    )(h_pad, d0, d1)


def _tc_final_body(h_ref, g_ref, acc_ref, dis_ref, c1_ref, c2_ref, o_ref):
    accsum = acc_ref[0] + acc_ref[1]
    a = dis_ref[...] * (accsum + g_ref[...])
    o_ref[...] = h_ref[...] * c1_ref[...] + a * c2_ref[...]


def _tc_final(h_pad, g_pad, acc, dis_col, c1, c2):
    grid = (NROWS // _RB,)
    return pl.pallas_call(
        _tc_final_body,
        grid=grid,
        in_specs=[
            pl.BlockSpec((_RB, D), lambda i: (i, 0)),
            pl.BlockSpec((_RB, D), lambda i: (i, 0)),
            pl.BlockSpec((2, _RB, D), lambda i: (0, i, 0)),
            pl.BlockSpec((_RB, 1), lambda i: (i, 0)),
            pl.BlockSpec((1, D), lambda i: (0, 0)),
            pl.BlockSpec((1, D), lambda i: (0, 0)),
        ],
        out_specs=pl.BlockSpec((_RB, D), lambda i: (i, 0)),
        out_shape=jax.ShapeDtypeStruct((NROWS, D), jnp.float32),
    )(h_pad, g_pad, acc, dis_col, c1, c2)


def kernel(h, edge_index, k_cross_channel, K_channel_wise):
    src = edge_index[0].astype(jnp.int32)
    dst = edge_index[1].astype(jnp.int32)
    pad = N + jnp.arange(EPAD - E, dtype=jnp.int32) % (NROWS - N)
    srcp = jnp.concatenate([src, pad]).reshape(NCH, EPB)
    dstp = jnp.concatenate([dst, pad]).reshape(NCH, EPB)
    h_pad = jnp.pad(h, ((0, NROWS - N), (0, 0)))

    deg_flat = _sc_deg(dstp)
    d0 = deg_flat[:NROWS, None]
    d1 = deg_flat[NDEG:NDEG + NROWS, None]
    g_pad, dis_col = _tc_g(h_pad, d0, d1)
    acc = _sc_spmm(g_pad, srcp, dstp)

    k = k_cross_channel[0]
    c1 = (2.0 - k - K_channel_wise) * 0.5
    c2 = (k + K_channel_wise) * 0.5
    out_pad = _tc_final(h_pad, g_pad, acc, dis_col, c1, c2)
    return out_pad[:N]


# speedup vs baseline: 3.2810x; 1.2147x over previous
import functools

import jax
import jax.numpy as jnp
from jax import lax
from jax.experimental import pallas as pl
from jax.experimental.pallas import tpu as pltpu
from jax.experimental.pallas import tpu_sc as plsc

N = 10000
D = 128
E = 320000

NW = 32
EPB = 128
CPT = 80
NCH = NW * CPT
EPAD = NCH * EPB
NROWS = 10112
RPT = NROWS // 16
NDEG = 10240
DPT = NDEG // 16

_mesh = plsc.VectorSubcoreMesh(core_axis_name="c", subcore_axis_name="s")


@functools.partial(
    pl.kernel,
    mesh=_mesh,
    out_type=jax.ShapeDtypeStruct((2 * NDEG,), jnp.float32),
    scratch_types=[
        pltpu.VMEM((NCH // NW, EPB), jnp.int32),
        pltpu.VMEM((EPB,), jnp.float32),
        pltpu.VMEM((DPT,), jnp.float32),
        pltpu.VMEM_SHARED((NDEG,), jnp.float32),
    ],
)
def _sc_deg(dst_hbm, out_hbm, idx_v, ones_v, zb_v, deg_sh):
    c = lax.axis_index("c")
    s = lax.axis_index("s")
    wid = c * 16 + s

    for i in range(EPB // 16):
        ones_v[pl.ds(i * 16, 16)] = jnp.ones((16,), jnp.float32)
    for i in range(DPT // 16):
        zb_v[pl.ds(i * 16, 16)] = jnp.zeros((16,), jnp.float32)
    pltpu.sync_copy(zb_v, deg_sh.at[pl.ds(s * DPT, DPT)])
    plsc.subcore_barrier()
    pltpu.sync_copy(dst_hbm.at[pl.ds(wid * CPT, CPT)], idx_v)

    def body(j, carry):
        pltpu.sync_copy(ones_v, deg_sh.at[idx_v.at[j]], add=True)
        return carry

    lax.fori_loop(0, CPT, body, 0)
    plsc.subcore_barrier()
    pltpu.sync_copy(deg_sh.at[pl.ds(s * DPT, DPT)],
                    out_hbm.at[pl.ds(c * NDEG + s * DPT, DPT)])


@functools.partial(
    pl.kernel,
    mesh=_mesh,
    out_type=jax.ShapeDtypeStruct((2, NROWS, D), jnp.float32),
    scratch_types=[
        pltpu.VMEM((CPT // 2, EPB), jnp.int32),
        pltpu.VMEM((CPT // 2, EPB), jnp.int32),
        pltpu.VMEM((EPB, D), jnp.float32),
        pltpu.VMEM((EPB, D), jnp.float32),
        pltpu.VMEM_SHARED((NROWS, D), jnp.float32),
        pltpu.SemaphoreType.DMA,
        pltpu.SemaphoreType.DMA,
        pltpu.SemaphoreType.DMA,
        pltpu.SemaphoreType.DMA,
    ],
)
def _sc_spmm(g_hbm, src_hbm, dst_hbm, out_hbm, si_v, di_v, rows_v, buf_b,
             acc_sh, sem, sem_b, sem_sa, sem_sb):
    c = lax.axis_index("c")
    s = lax.axis_index("s")
    wid = c * 16 + s

    def zrow(j, carry):
        for i in range(D // 16):
            rows_v[j, pl.ds(i * 16, 16)] = jnp.zeros((16,), jnp.float32)
        return carry

    lax.fori_loop(0, EPB, zrow, 0)
    for b in range(4):
        pltpu.sync_copy(rows_v, acc_sh.at[pl.ds(s * RPT + b * EPB, EPB)])
    pltpu.sync_copy(rows_v.at[pl.ds(0, RPT - 4 * EPB)],
                    acc_sh.at[pl.ds(s * RPT + 4 * EPB, RPT - 4 * EPB)])
    plsc.subcore_barrier()
    H = CPT // 2
    for half in range(2):
        pltpu.sync_copy(src_hbm.at[pl.ds(wid * CPT + half * H, H)], si_v)
        pltpu.sync_copy(dst_hbm.at[pl.ds(wid * CPT + half * H, H)], di_v)
        pltpu.async_copy(g_hbm.at[di_v.at[0]], rows_v, sem).wait()
        pltpu.async_copy(rows_v, acc_sh.at[si_v.at[0]], sem_sa, add=True)
        pltpu.async_copy(g_hbm.at[di_v.at[1]], buf_b, sem_b).wait()
        pltpu.async_copy(buf_b, acc_sh.at[si_v.at[1]], sem_sb, add=True)

        def body(i, carry):
            j = 2 * i + 2
            pltpu.make_async_copy(
                rows_v, acc_sh.at[si_v.at[j - 2]], sem_sa).wait()
            pltpu.async_copy(g_hbm.at[di_v.at[j]], rows_v, sem).wait()
            pltpu.async_copy(rows_v, acc_sh.at[si_v.at[j]], sem_sa, add=True)
            pltpu.make_async_copy(
                buf_b, acc_sh.at[si_v.at[j - 1]], sem_sb).wait()
            pltpu.async_copy(g_hbm.at[di_v.at[j + 1]], buf_b, sem_b).wait()
            pltpu.async_copy(
                buf_b, acc_sh.at[si_v.at[j + 1]], sem_sb, add=True)
            return carry

        lax.fori_loop(0, H // 2 - 1, body, 0)
        pltpu.make_async_copy(
            rows_v, acc_sh.at[si_v.at[H - 2]], sem_sa).wait()
        pltpu.make_async_copy(
            buf_b, acc_sh.at[si_v.at[H - 1]], sem_sb).wait()
    plsc.subcore_barrier()
    pltpu.sync_copy(acc_sh.at[pl.ds(s * RPT, RPT)],
                    out_hbm.at[c, pl.ds(s * RPT, RPT)])


_RB = 2528


def _tc_g_body(h_ref, d0_ref, d1_ref, g_ref, dis_ref):
    dis = lax.rsqrt(d0_ref[...] + d1_ref[...] + 1.0)
    g_ref[...] = h_ref[...] * dis
    dis_ref[...] = dis


def _tc_g(h_pad, d0, d1):
    grid = (NROWS // _RB,)
    return pl.pallas_call(
        _tc_g_body,
        grid=grid,
        in_specs=[
            pl.BlockSpec((_RB, D), lambda i: (i, 0)),
            pl.BlockSpec((_RB, 1), lambda i: (i, 0)),
            pl.BlockSpec((_RB, 1), lambda i: (i, 0)),
        ],
        out_specs=[
            pl.BlockSpec((_RB, D), lambda i: (i, 0)),
            pl.BlockSpec((_RB, 1), lambda i: (i, 0)),
        ],
        out_shape=[
            jax.ShapeDtypeStruct((NROWS, D), jnp.float32),
            jax.ShapeDtypeStruct((NROWS, 1), jnp.float32),
        ],
    )(h_pad, d0, d1)


def _tc_final_body(h_ref, g_ref, acc_ref, dis_ref, c1_ref, c2_ref, o_ref):
    accsum = acc_ref[0] + acc_ref[1]
    a = dis_ref[...] * (accsum + g_ref[...])
    o_ref[...] = h_ref[...] * c1_ref[...] + a * c2_ref[...]


def _tc_final(h_pad, g_pad, acc, dis_col, c1, c2):
    grid = (NROWS // _RB,)
    return pl.pallas_call(
        _tc_final_body,
        grid=grid,
        in_specs=[
            pl.BlockSpec((_RB, D), lambda i: (i, 0)),
            pl.BlockSpec((_RB, D), lambda i: (i, 0)),
            pl.BlockSpec((2, _RB, D), lambda i: (0, i, 0)),
            pl.BlockSpec((_RB, 1), lambda i: (i, 0)),
            pl.BlockSpec((1, D), lambda i: (0, 0)),
            pl.BlockSpec((1, D), lambda i: (0, 0)),
        ],
        out_specs=pl.BlockSpec((_RB, D), lambda i: (i, 0)),
        out_shape=jax.ShapeDtypeStruct((NROWS, D), jnp.float32),
    )(h_pad, g_pad, acc, dis_col, c1, c2)


def kernel(h, edge_index, k_cross_channel, K_channel_wise):
    src = edge_index[0].astype(jnp.int32)
    dst = edge_index[1].astype(jnp.int32)
    pad = N + jnp.arange(EPAD - E, dtype=jnp.int32) % (NROWS - N)
    srcp = jnp.concatenate([src, pad]).reshape(NCH, EPB)
    dstp = jnp.concatenate([dst, pad]).reshape(NCH, EPB)
    h_pad = jnp.pad(h, ((0, NROWS - N), (0, 0)))

    deg_flat = _sc_deg(dstp)
    d0 = deg_flat[:NROWS, None]
    d1 = deg_flat[NDEG:NDEG + NROWS, None]
    g_pad, dis_col = _tc_g(h_pad, d0, d1)
    acc = _sc_spmm(g_pad, srcp, dstp)

    k = k_cross_channel[0]
    c1 = (2.0 - k - K_channel_wise) * 0.5
    c2 = (k + K_channel_wise) * 0.5
    out_pad = _tc_final(h_pad, g_pad, acc, dis_col, c1, c2)
    return out_pad[:N]


# speedup vs baseline: 3.3432x; 1.0190x over previous
import functools

import jax
import jax.numpy as jnp
from jax import lax
from jax.experimental import pallas as pl
from jax.experimental.pallas import tpu as pltpu
from jax.experimental.pallas import tpu_sc as plsc

N = 10000
D = 128
E = 320000

NW = 32
EPB = 128
CPT = 80
NCH = NW * CPT
EPAD = NCH * EPB
NROWS = 10112
RPT = NROWS // 16
NDEG = 10240
DPT = NDEG // 16

_mesh = plsc.VectorSubcoreMesh(core_axis_name="c", subcore_axis_name="s")


@functools.partial(
    pl.kernel,
    mesh=_mesh,
    out_type=jax.ShapeDtypeStruct((2 * NDEG,), jnp.float32),
    scratch_types=[
        pltpu.VMEM((NCH // NW, EPB), jnp.int32),
        pltpu.VMEM((EPB,), jnp.float32),
        pltpu.VMEM((DPT,), jnp.float32),
        pltpu.VMEM_SHARED((NDEG,), jnp.float32),
    ],
)
def _sc_deg(dst_hbm, out_hbm, idx_v, ones_v, zb_v, deg_sh):
    c = lax.axis_index("c")
    s = lax.axis_index("s")
    wid = c * 16 + s

    for i in range(EPB // 16):
        ones_v[pl.ds(i * 16, 16)] = jnp.ones((16,), jnp.float32)
    for i in range(DPT // 16):
        zb_v[pl.ds(i * 16, 16)] = jnp.zeros((16,), jnp.float32)
    pltpu.sync_copy(zb_v, deg_sh.at[pl.ds(s * DPT, DPT)])
    plsc.subcore_barrier()
    pltpu.sync_copy(dst_hbm.at[pl.ds(wid * CPT, CPT)], idx_v)

    def body(j, carry):
        pltpu.sync_copy(ones_v, deg_sh.at[idx_v.at[j]], add=True)
        return carry

    lax.fori_loop(0, CPT, body, 0)
    plsc.subcore_barrier()
    pltpu.sync_copy(deg_sh.at[pl.ds(s * DPT, DPT)],
                    out_hbm.at[pl.ds(c * NDEG + s * DPT, DPT)])


@functools.partial(
    pl.kernel,
    mesh=_mesh,
    out_type=jax.ShapeDtypeStruct((2, NROWS, D), jnp.float32),
    scratch_types=[
        pltpu.VMEM((CPT // 2, EPB), jnp.int32),
        pltpu.VMEM((CPT // 2, EPB), jnp.int32),
        pltpu.VMEM((EPB, D), jnp.float32),
        pltpu.VMEM((EPB, D), jnp.float32),
        pltpu.VMEM_SHARED((NROWS, D), jnp.float32),
        pltpu.SemaphoreType.DMA,
        pltpu.SemaphoreType.DMA,
        pltpu.SemaphoreType.DMA,
        pltpu.SemaphoreType.DMA,
    ],
)
def _sc_spmm(g_hbm, src_hbm, dst_hbm, out_hbm, si_v, di_v, rows_v, buf_b,
             acc_sh, sem, sem_b, sem_sa, sem_sb):
    c = lax.axis_index("c")
    s = lax.axis_index("s")
    wid = c * 16 + s

    def zrow(j, carry):
        for i in range(D // 16):
            rows_v[j, pl.ds(i * 16, 16)] = jnp.zeros((16,), jnp.float32)
        return carry

    lax.fori_loop(0, EPB, zrow, 0)
    for b in range(4):
        pltpu.sync_copy(rows_v, acc_sh.at[pl.ds(s * RPT + b * EPB, EPB)])
    pltpu.sync_copy(rows_v.at[pl.ds(0, RPT - 4 * EPB)],
                    acc_sh.at[pl.ds(s * RPT + 4 * EPB, RPT - 4 * EPB)])
    plsc.subcore_barrier()
    H = CPT // 2
    for half in range(2):
        pltpu.sync_copy(src_hbm.at[pl.ds(wid * CPT + half * H, H)], si_v)
        pltpu.sync_copy(dst_hbm.at[pl.ds(wid * CPT + half * H, H)], di_v)
        pltpu.async_copy(g_hbm.at[di_v.at[0]], rows_v, sem).wait()
        pltpu.async_copy(rows_v, acc_sh.at[si_v.at[0]], sem_sa, add=True)
        pltpu.async_copy(g_hbm.at[di_v.at[1]], buf_b, sem_b).wait()
        pltpu.async_copy(buf_b, acc_sh.at[si_v.at[1]], sem_sb, add=True)

        def body(i, carry):
            j = 2 * i + 2
            pltpu.make_async_copy(
                rows_v, acc_sh.at[si_v.at[j - 2]], sem_sa).wait()
            pltpu.async_copy(g_hbm.at[di_v.at[j]], rows_v, sem).wait()
            pltpu.async_copy(rows_v, acc_sh.at[si_v.at[j]], sem_sa, add=True)
            pltpu.make_async_copy(
                buf_b, acc_sh.at[si_v.at[j - 1]], sem_sb).wait()
            pltpu.async_copy(g_hbm.at[di_v.at[j + 1]], buf_b, sem_b).wait()
            pltpu.async_copy(
                buf_b, acc_sh.at[si_v.at[j + 1]], sem_sb, add=True)
            return carry

        lax.fori_loop(0, H // 2 - 1, body, 0)
        pltpu.make_async_copy(
            rows_v, acc_sh.at[si_v.at[H - 2]], sem_sa).wait()
        pltpu.make_async_copy(
            buf_b, acc_sh.at[si_v.at[H - 1]], sem_sb).wait()
    plsc.subcore_barrier()
    pltpu.sync_copy(acc_sh.at[pl.ds(s * RPT, RPT)],
                    out_hbm.at[c, pl.ds(s * RPT, RPT)])


_RB = 2528


def _tc_g_body(h_ref, d0_ref, d1_ref, g_ref, dis_ref):
    dis = lax.rsqrt(d0_ref[...] + d1_ref[...] + 1.0)
    g_ref[...] = h_ref[...] * dis
    dis_ref[...] = dis


def _tc_g(h_pad, d0, d1):
    grid = (NROWS // _RB,)
    return pl.pallas_call(
        _tc_g_body,
        grid=grid,
        in_specs=[
            pl.BlockSpec((_RB, D), lambda i: (i, 0)),
            pl.BlockSpec((_RB, 1), lambda i: (i, 0)),
            pl.BlockSpec((_RB, 1), lambda i: (i, 0)),
        ],
        out_specs=[
            pl.BlockSpec((_RB, D), lambda i: (i, 0)),
            pl.BlockSpec((_RB, 1), lambda i: (i, 0)),
        ],
        out_shape=[
            jax.ShapeDtypeStruct((NROWS, D), jnp.float32),
            jax.ShapeDtypeStruct((NROWS, 1), jnp.float32),
        ],
    )(h_pad, d0, d1)


def _tc_final_body(h_ref, g_ref, acc_ref, dis_ref, c1_ref, c2_ref, o_ref):
    accsum = acc_ref[0] + acc_ref[1]
    a = dis_ref[...] * (accsum + g_ref[...])
    o_ref[...] = h_ref[...] * c1_ref[...] + a * c2_ref[...]


def _tc_final(h_pad, g_pad, acc, dis_col, c1, c2):
    rb = N // 5
    return pl.pallas_call(
        _tc_final_body,
        grid=(5,),
        in_specs=[
            pl.BlockSpec((rb, D), lambda i: (i, 0)),
            pl.BlockSpec((rb, D), lambda i: (i, 0)),
            pl.BlockSpec((2, rb, D), lambda i: (0, i, 0)),
            pl.BlockSpec((rb, 1), lambda i: (i, 0)),
            pl.BlockSpec((1, D), lambda i: (0, 0)),
            pl.BlockSpec((1, D), lambda i: (0, 0)),
        ],
        out_specs=pl.BlockSpec((rb, D), lambda i: (i, 0)),
        out_shape=jax.ShapeDtypeStruct((N, D), jnp.float32),
    )(h_pad, g_pad, acc, dis_col, c1, c2)


def kernel(h, edge_index, k_cross_channel, K_channel_wise):
    src = edge_index[0].astype(jnp.int32)
    dst = edge_index[1].astype(jnp.int32)
    pad = N + jnp.arange(EPAD - E, dtype=jnp.int32) % (NROWS - N)
    srcp = jnp.concatenate([src, pad]).reshape(NCH, EPB)
    dstp = jnp.concatenate([dst, pad]).reshape(NCH, EPB)
    h_pad = jnp.pad(h, ((0, NROWS - N), (0, 0)))

    deg_flat = _sc_deg(dstp)
    d0 = deg_flat[:NROWS, None]
    d1 = deg_flat[NDEG:NDEG + NROWS, None]
    g_pad, dis_col = _tc_g(h_pad, d0, d1)
    acc = _sc_spmm(g_pad, srcp, dstp)

    k = k_cross_channel[0]
    c1 = (2.0 - k - K_channel_wise) * 0.5
    c2 = (k + K_channel_wise) * 0.5
    return _tc_final(h_pad, g_pad, acc, dis_col, c1, c2)
